# Initial kernel scaffold; baseline (speedup 1.0000x reference)
import jax
import jax.numpy as jnp
from jax.experimental import pallas as pl


def kernel(x_cont, x_cat, emb, gamma_c, beta_c, W1, b1, g1, bt1, W2, b2, g2, bt2, W3, b3):
    raise NotImplementedError("write your pallas kernel here")


# speedup vs baseline: 7.3774x; 7.3774x over previous
import functools

import jax
import jax.numpy as jnp
from jax import lax
from jax.experimental import pallas as pl
from jax.experimental.pallas import tpu as pltpu
from jax.experimental.pallas import tpu_sc as plsc

_B = 16384
_F = 26
_V = 100000
_D = 16
_C = 13
_H = 512
_O = 10
_EPS = 1e-5

_NC, _NS = 2, 16
_NW = _NC * _NS
_N_ROWS = _B * _F
_GPW = _N_ROWS // (_NW * 128)
_K = 13
_STEPS = _GPW // _K


def _sc_gather(table, idx2d):
    mesh = plsc.VectorSubcoreMesh(core_axis_name="c", subcore_axis_name="s")

    @functools.partial(
        pl.kernel,
        out_type=jax.ShapeDtypeStruct((_N_ROWS, _D), jnp.float32),
        mesh=mesh,
        scratch_types=[
            pltpu.VMEM((_GPW, 128), jnp.int32),
            pltpu.VMEM((_K * 128, _D), jnp.float32),
            pltpu.SemaphoreType.DMA,
        ],
        compiler_params=pltpu.CompilerParams(use_tc_tiling_on_sc=False),
    )
    def k(table_hbm, idx_hbm, out_hbm, idx_v, rows_v, sem):
        wid = lax.axis_index("s") * _NC + lax.axis_index("c")
        gbase = wid * _GPW
        pltpu.sync_copy(idx_hbm.at[pl.ds(gbase, _GPW)], idx_v)

        def step(s, _):
            handles = []
            for j in range(_K):
                g = s * _K + j
                handles.append(
                    pltpu.async_copy(
                        table_hbm.at[idx_v.at[g]],
                        rows_v.at[pl.ds(j * 128, 128)],
                        sem,
                    )
                )
            for h in handles:
                h.wait()
            out_off = (gbase + s * _K) * 128
            pltpu.sync_copy(rows_v, out_hbm.at[pl.ds(out_off, _K * 128)])
            return ()

        lax.fori_loop(0, _STEPS, step, (), unroll=False)

    return k(table, idx2d)


_R = 512
_G = _B // _R


def _l1_body(e_ref, xc_ref, gc_ref, bc_ref, w1e_ref, w1c_ref, b1_ref,
             a1_ref, s1_ref, stat_ref):
    i = pl.program_id(0)

    @pl.when(i == 0)
    def _():
        xc = xc_ref[...]
        m = jnp.mean(xc, axis=0, keepdims=True)
        v = jnp.mean(xc * xc, axis=0, keepdims=True) - m * m
        scale = gc_ref[...] * lax.rsqrt(v + _EPS)
        shift = bc_ref[...] - m * scale
        stat_ref[0:1, :] = scale
        stat_ref[1:2, :] = shift

    xcn = xc_ref[pl.ds(i * _R, _R), :] * stat_ref[0:1, :] + stat_ref[1:2, :]
    h = jnp.dot(e_ref[...], w1e_ref[...], preferred_element_type=jnp.float32)
    h += jnp.dot(xcn, w1c_ref[...], preferred_element_type=jnp.float32)
    a1 = jnp.maximum(h + b1_ref[...], 0.0)
    a1_ref[...] = a1

    @pl.when(i == 0)
    def _():
        s1_ref[...] = jnp.zeros_like(s1_ref)

    s1_ref[0:1, :] += jnp.sum(a1, axis=0, keepdims=True)
    s1_ref[1:2, :] += jnp.sum(a1 * a1, axis=0, keepdims=True)


def _l2_body(a1_ref, s1_ref, g1_ref, bt1_ref, w2_ref, b2_ref,
             a2_ref, s2_ref):
    i = pl.program_id(0)
    m = s1_ref[0:1, :] * (1.0 / _B)
    v = s1_ref[1:2, :] * (1.0 / _B) - m * m
    scale = g1_ref[...] * lax.rsqrt(v + _EPS)
    shift = bt1_ref[...] - m * scale
    a1n = a1_ref[...] * scale + shift
    a2 = jnp.maximum(
        jnp.dot(a1n, w2_ref[...], preferred_element_type=jnp.float32)
        + b2_ref[...], 0.0)
    a2_ref[...] = a2

    @pl.when(i == 0)
    def _():
        s2_ref[...] = jnp.zeros_like(s2_ref)

    s2_ref[0:1, :] += jnp.sum(a2, axis=0, keepdims=True)
    s2_ref[1:2, :] += jnp.sum(a2 * a2, axis=0, keepdims=True)


def _l3_body(a2_ref, s2_ref, g2_ref, bt2_ref, w3_ref, b3_ref, out_ref):
    m = s2_ref[0:1, :] * (1.0 / _B)
    v = s2_ref[1:2, :] * (1.0 / _B) - m * m
    scale = g2_ref[...] * lax.rsqrt(v + _EPS)
    shift = bt2_ref[...] - m * scale
    a2n = a2_ref[...] * scale + shift
    out_ref[...] = (
        jnp.dot(a2n, w3_ref[...], preferred_element_type=jnp.float32)
        + b3_ref[...])


def _row(x):
    return x.reshape(1, -1)


def kernel(x_cont, x_cat, emb, gamma_c, beta_c, W1, b1, g1, bt1,
           W2, b2, g2, bt2, W3, b3):
    table = emb.reshape(_F * _V, _D)
    flat_idx = (x_cat + (jnp.arange(_F, dtype=jnp.int32) * _V)[None, :])
    idx2d = flat_idx.reshape(_N_ROWS // 128, 128)

    e = _sc_gather(table, idx2d).reshape(_B, _F * _D)

    full = lambda s: pl.BlockSpec(s, lambda i: (0, 0))
    blk = lambda r, c: pl.BlockSpec((r, c), lambda i: (i, 0))

    a1, s1 = pl.pallas_call(
        _l1_body,
        grid=(_G,),
        in_specs=[
            blk(_R, _F * _D),
            full((_B, _C)),
            full((1, _C)),
            full((1, _C)),
            full((_F * _D, _H)),
            full((_C, _H)),
            full((1, _H)),
        ],
        out_specs=[blk(_R, _H), full((2, _H))],
        out_shape=[
            jax.ShapeDtypeStruct((_B, _H), jnp.float32),
            jax.ShapeDtypeStruct((2, _H), jnp.float32),
        ],
        scratch_shapes=[pltpu.VMEM((2, _C), jnp.float32)],
    )(e, x_cont, _row(gamma_c), _row(beta_c), W1[:_F * _D], W1[_F * _D:],
      _row(b1))

    a2, s2 = pl.pallas_call(
        _l2_body,
        grid=(_G,),
        in_specs=[
            blk(_R, _H),
            full((2, _H)),
            full((1, _H)),
            full((1, _H)),
            full((_H, _H // 2)),
            full((1, _H // 2)),
        ],
        out_specs=[blk(_R, _H // 2), full((2, _H // 2))],
        out_shape=[
            jax.ShapeDtypeStruct((_B, _H // 2), jnp.float32),
            jax.ShapeDtypeStruct((2, _H // 2), jnp.float32),
        ],
    )(a1, s1, _row(g1), _row(bt1), W2, _row(b2))

    out = pl.pallas_call(
        _l3_body,
        grid=(_G,),
        in_specs=[
            blk(_R, _H // 2),
            full((2, _H // 2)),
            full((1, _H // 2)),
            full((1, _H // 2)),
            full((_H // 2, _O)),
            full((1, _O)),
        ],
        out_specs=blk(_R, _O),
        out_shape=jax.ShapeDtypeStruct((_B, _O), jnp.float32),
    )(a2, s2, _row(g2), _row(bt2), W3, _row(b3))

    return out


# speedup vs baseline: 26.5272x; 3.5957x over previous
import functools

import jax
import jax.numpy as jnp
from jax import lax
from jax.experimental import pallas as pl
from jax.experimental.pallas import tpu as pltpu
from jax.experimental.pallas import tpu_sc as plsc

_B = 16384
_F = 26
_V = 100000
_D = 16
_C = 13
_H = 512
_O = 10
_EPS = 1e-5

_NC, _NS = 2, 16
_NW = _NC * _NS
_NR = _F * _D
_RPW = _NR // _NW
_HB = _B // 2


def _sc_gather_t(table, idx_t):
    mesh = plsc.VectorSubcoreMesh(core_axis_name="c", subcore_axis_name="s")

    @functools.partial(
        pl.kernel,
        out_type=jax.ShapeDtypeStruct((_NR, _B), jnp.float32),
        mesh=mesh,
        scratch_types=[
            pltpu.VMEM((_V,), jnp.float32),
            pltpu.VMEM((_B,), jnp.int32),
            pltpu.VMEM((_HB,), jnp.float32),
        ],
        compiler_params=pltpu.CompilerParams(
            use_tc_tiling_on_sc=True, needs_layout_passes=False),
    )
    def k(table_hbm, idx_hbm, out_hbm, row_v, idx_v, out_v):
        wid = lax.axis_index("s") * _NC + lax.axis_index("c")

        def do_row(r, _):
            j = wid * _RPW + r
            f = j // _D
            pltpu.sync_copy(table_hbm.at[j], row_v)
            pltpu.sync_copy(idx_hbm.at[f], idx_v)

            def do_half(h, _):
                def gat(i, _):
                    iv = idx_v[pl.ds(h * _HB + i * 16, 16)]
                    out_v[pl.ds(i * 16, 16)] = plsc.load_gather(row_v, [iv])
                    return ()

                lax.fori_loop(0, _HB // 16, gat, (), unroll=8)
                pltpu.sync_copy(out_v, out_hbm.at[j, pl.ds(h * _HB, _HB)])
                return ()

            lax.fori_loop(0, 2, do_half, (), unroll=True)
            return ()

        lax.fori_loop(0, _RPW, do_row, (), unroll=False)

    return k(table, idx_t)


_R = 512
_G = _B // _R


def _l1_body(et_ref, xc_ref, gc_ref, bc_ref, w1e_ref, w1c_ref, b1_ref,
             a1_ref, s1_ref, stat_ref):
    i = pl.program_id(0)

    @pl.when(i == 0)
    def _():
        xc = xc_ref[...]
        m = jnp.mean(xc, axis=0, keepdims=True)
        v = jnp.mean(xc * xc, axis=0, keepdims=True) - m * m
        scale = gc_ref[...] * lax.rsqrt(v + _EPS)
        shift = bc_ref[...] - m * scale
        stat_ref[0:1, :] = scale
        stat_ref[1:2, :] = shift

    xcn = xc_ref[pl.ds(i * _R, _R), :] * stat_ref[0:1, :] + stat_ref[1:2, :]
    h = lax.dot_general(et_ref[...], w1e_ref[...], (((0,), (0,)), ((), ())),
                        preferred_element_type=jnp.float32)
    h += jnp.dot(xcn, w1c_ref[...], preferred_element_type=jnp.float32)
    a1 = jnp.maximum(h + b1_ref[...], 0.0)
    a1_ref[...] = a1

    @pl.when(i == 0)
    def _():
        s1_ref[...] = jnp.zeros_like(s1_ref)

    s1_ref[0:1, :] += jnp.sum(a1, axis=0, keepdims=True)
    s1_ref[1:2, :] += jnp.sum(a1 * a1, axis=0, keepdims=True)


def _l2_body(a1_ref, s1_ref, g1_ref, bt1_ref, w2_ref, b2_ref,
             a2_ref, s2_ref):
    i = pl.program_id(0)
    m = s1_ref[0:1, :] * (1.0 / _B)
    v = s1_ref[1:2, :] * (1.0 / _B) - m * m
    scale = g1_ref[...] * lax.rsqrt(v + _EPS)
    shift = bt1_ref[...] - m * scale
    a1n = a1_ref[...] * scale + shift
    a2 = jnp.maximum(
        jnp.dot(a1n, w2_ref[...], preferred_element_type=jnp.float32)
        + b2_ref[...], 0.0)
    a2_ref[...] = a2

    @pl.when(i == 0)
    def _():
        s2_ref[...] = jnp.zeros_like(s2_ref)

    s2_ref[0:1, :] += jnp.sum(a2, axis=0, keepdims=True)
    s2_ref[1:2, :] += jnp.sum(a2 * a2, axis=0, keepdims=True)


def _l3_body(a2_ref, s2_ref, g2_ref, bt2_ref, w3_ref, b3_ref, out_ref):
    m = s2_ref[0:1, :] * (1.0 / _B)
    v = s2_ref[1:2, :] * (1.0 / _B) - m * m
    scale = g2_ref[...] * lax.rsqrt(v + _EPS)
    shift = bt2_ref[...] - m * scale
    a2n = a2_ref[...] * scale + shift
    out_ref[...] = (
        jnp.dot(a2n, w3_ref[...], preferred_element_type=jnp.float32)
        + b3_ref[...])


def _row(x):
    return x.reshape(1, -1)


def kernel(x_cont, x_cat, emb, gamma_c, beta_c, W1, b1, g1, bt1,
           W2, b2, g2, bt2, W3, b3):
    table = emb.transpose(0, 2, 1).reshape(_NR, _V)
    idx_t = x_cat.T

    e_t = _sc_gather_t(table, idx_t)

    full = lambda s: pl.BlockSpec(s, lambda i: (0, 0))
    blk = lambda r, c: pl.BlockSpec((r, c), lambda i: (i, 0))
    cblk = lambda r, c: pl.BlockSpec((r, c), lambda i: (0, i))

    a1, s1 = pl.pallas_call(
        _l1_body,
        grid=(_G,),
        in_specs=[
            cblk(_NR, _R),
            full((_B, _C)),
            full((1, _C)),
            full((1, _C)),
            full((_NR, _H)),
            full((_C, _H)),
            full((1, _H)),
        ],
        out_specs=[blk(_R, _H), full((2, _H))],
        out_shape=[
            jax.ShapeDtypeStruct((_B, _H), jnp.float32),
            jax.ShapeDtypeStruct((2, _H), jnp.float32),
        ],
        scratch_shapes=[pltpu.VMEM((2, _C), jnp.float32)],
    )(e_t, x_cont, _row(gamma_c), _row(beta_c), W1[:_NR], W1[_NR:],
      _row(b1))

    a2, s2 = pl.pallas_call(
        _l2_body,
        grid=(_G,),
        in_specs=[
            blk(_R, _H),
            full((2, _H)),
            full((1, _H)),
            full((1, _H)),
            full((_H, _H // 2)),
            full((1, _H // 2)),
        ],
        out_specs=[blk(_R, _H // 2), full((2, _H // 2))],
        out_shape=[
            jax.ShapeDtypeStruct((_B, _H // 2), jnp.float32),
            jax.ShapeDtypeStruct((2, _H // 2), jnp.float32),
        ],
    )(a1, s1, _row(g1), _row(bt1), W2, _row(b2))

    out = pl.pallas_call(
        _l3_body,
        grid=(_G,),
        in_specs=[
            blk(_R, _H // 2),
            full((2, _H // 2)),
            full((1, _H // 2)),
            full((1, _H // 2)),
            full((_H // 2, _O)),
            full((1, _O)),
        ],
        out_specs=blk(_R, _O),
        out_shape=jax.ShapeDtypeStruct((_B, _O), jnp.float32),
    )(a2, s2, _row(g2), _row(bt2), W3, _row(b3))

    return out


# speedup vs baseline: 37.4182x; 1.4106x over previous
import functools

import jax
import jax.numpy as jnp
from jax import lax
from jax.experimental import pallas as pl
from jax.experimental.pallas import tpu as pltpu
from jax.experimental.pallas import tpu_sc as plsc

_B = 16384
_F = 26
_V = 100000
_D = 16
_C = 13
_H = 512
_O = 10
_EPS = 1e-5

_NC, _NS = 2, 16
_NW = _NC * _NS
_NR = _F * _D
_RPW = _NR // _NW
_HB = _B // 2


def _sc_gather_t(table, idx_t):
    mesh = plsc.VectorSubcoreMesh(core_axis_name="c", subcore_axis_name="s")

    @functools.partial(
        pl.kernel,
        out_type=jax.ShapeDtypeStruct((_NR, _B), jnp.float32),
        mesh=mesh,
        scratch_types=[
            pltpu.VMEM((_V,), jnp.float32),
            pltpu.VMEM((_B,), jnp.int32),
            pltpu.VMEM((_HB,), jnp.float32),
        ],
        compiler_params=pltpu.CompilerParams(
            use_tc_tiling_on_sc=True, needs_layout_passes=False),
    )
    def k(table_hbm, idx_hbm, out_hbm, row_v, idx_v, out_v):
        wid = lax.axis_index("s") * _NC + lax.axis_index("c")

        def do_row(r, _):
            j = wid * _RPW + r
            f = j // _D
            pltpu.sync_copy(table_hbm.at[j], row_v)
            pltpu.sync_copy(idx_hbm.at[f], idx_v)

            def do_half(h, _):
                @plsc.parallel_loop(0, _HB, step=16, unroll=8)
                def gat(i):
                    iv = idx_v[pl.ds(h * _HB + i, 16)]
                    out_v[pl.ds(i, 16)] = plsc.load_gather(row_v, [iv])

                pltpu.sync_copy(out_v, out_hbm.at[j, pl.ds(h * _HB, _HB)])
                return ()

            lax.fori_loop(0, 2, do_half, (), unroll=True)
            return ()

        lax.fori_loop(0, _RPW, do_row, (), unroll=False)

    return k(table, idx_t)


_R = 512
_G = _B // _R


def _l1_body(et_ref, xc_ref, gc_ref, bc_ref, w1e_ref, w1c_ref, b1_ref,
             a1_ref, s1_ref, stat_ref):
    i = pl.program_id(0)

    @pl.when(i == 0)
    def _():
        xc = xc_ref[...]
        m = jnp.mean(xc, axis=0, keepdims=True)
        v = jnp.mean(xc * xc, axis=0, keepdims=True) - m * m
        scale = gc_ref[...] * lax.rsqrt(v + _EPS)
        shift = bc_ref[...] - m * scale
        stat_ref[0:1, :] = scale
        stat_ref[1:2, :] = shift

    xcn = xc_ref[pl.ds(i * _R, _R), :] * stat_ref[0:1, :] + stat_ref[1:2, :]
    h = lax.dot_general(et_ref[...], w1e_ref[...], (((0,), (0,)), ((), ())),
                        preferred_element_type=jnp.float32)
    h += jnp.dot(xcn, w1c_ref[...], preferred_element_type=jnp.float32)
    a1 = jnp.maximum(h + b1_ref[...], 0.0)
    a1_ref[...] = a1

    @pl.when(i == 0)
    def _():
        s1_ref[...] = jnp.zeros_like(s1_ref)

    s1_ref[0:1, :] += jnp.sum(a1, axis=0, keepdims=True)
    s1_ref[1:2, :] += jnp.sum(a1 * a1, axis=0, keepdims=True)


def _l2_body(a1_ref, s1_ref, g1_ref, bt1_ref, w2_ref, b2_ref,
             a2_ref, s2_ref):
    i = pl.program_id(0)
    m = s1_ref[0:1, :] * (1.0 / _B)
    v = s1_ref[1:2, :] * (1.0 / _B) - m * m
    scale = g1_ref[...] * lax.rsqrt(v + _EPS)
    shift = bt1_ref[...] - m * scale
    a1n = a1_ref[...] * scale + shift
    a2 = jnp.maximum(
        jnp.dot(a1n, w2_ref[...], preferred_element_type=jnp.float32)
        + b2_ref[...], 0.0)
    a2_ref[...] = a2

    @pl.when(i == 0)
    def _():
        s2_ref[...] = jnp.zeros_like(s2_ref)

    s2_ref[0:1, :] += jnp.sum(a2, axis=0, keepdims=True)
    s2_ref[1:2, :] += jnp.sum(a2 * a2, axis=0, keepdims=True)


def _l3_body(a2_ref, s2_ref, g2_ref, bt2_ref, w3_ref, b3_ref, out_ref):
    m = s2_ref[0:1, :] * (1.0 / _B)
    v = s2_ref[1:2, :] * (1.0 / _B) - m * m
    scale = g2_ref[...] * lax.rsqrt(v + _EPS)
    shift = bt2_ref[...] - m * scale
    a2n = a2_ref[...] * scale + shift
    out_ref[...] = (
        jnp.dot(a2n, w3_ref[...], preferred_element_type=jnp.float32)
        + b3_ref[...])


def _row(x):
    return x.reshape(1, -1)


def kernel(x_cont, x_cat, emb, gamma_c, beta_c, W1, b1, g1, bt1,
           W2, b2, g2, bt2, W3, b3):
    table = emb.transpose(0, 2, 1).reshape(_NR, _V)
    idx_t = x_cat.T

    e_t = _sc_gather_t(table, idx_t)

    full = lambda s: pl.BlockSpec(s, lambda i: (0, 0))
    blk = lambda r, c: pl.BlockSpec((r, c), lambda i: (i, 0))
    cblk = lambda r, c: pl.BlockSpec((r, c), lambda i: (0, i))

    a1, s1 = pl.pallas_call(
        _l1_body,
        grid=(_G,),
        in_specs=[
            cblk(_NR, _R),
            full((_B, _C)),
            full((1, _C)),
            full((1, _C)),
            full((_NR, _H)),
            full((_C, _H)),
            full((1, _H)),
        ],
        out_specs=[blk(_R, _H), full((2, _H))],
        out_shape=[
            jax.ShapeDtypeStruct((_B, _H), jnp.float32),
            jax.ShapeDtypeStruct((2, _H), jnp.float32),
        ],
        scratch_shapes=[pltpu.VMEM((2, _C), jnp.float32)],
    )(e_t, x_cont, _row(gamma_c), _row(beta_c), W1[:_NR], W1[_NR:],
      _row(b1))

    a2, s2 = pl.pallas_call(
        _l2_body,
        grid=(_G,),
        in_specs=[
            blk(_R, _H),
            full((2, _H)),
            full((1, _H)),
            full((1, _H)),
            full((_H, _H // 2)),
            full((1, _H // 2)),
        ],
        out_specs=[blk(_R, _H // 2), full((2, _H // 2))],
        out_shape=[
            jax.ShapeDtypeStruct((_B, _H // 2), jnp.float32),
            jax.ShapeDtypeStruct((2, _H // 2), jnp.float32),
        ],
    )(a1, s1, _row(g1), _row(bt1), W2, _row(b2))

    out = pl.pallas_call(
        _l3_body,
        grid=(_G,),
        in_specs=[
            blk(_R, _H // 2),
            full((2, _H // 2)),
            full((1, _H // 2)),
            full((1, _H // 2)),
            full((_H // 2, _O)),
            full((1, _O)),
        ],
        out_specs=blk(_R, _O),
        out_shape=jax.ShapeDtypeStruct((_B, _O), jnp.float32),
    )(a2, s2, _row(g2), _row(bt2), W3, _row(b3))

    return out


# speedup vs baseline: 37.5750x; 1.0042x over previous
import functools

import jax
import jax.numpy as jnp
from jax import lax
from jax.experimental import pallas as pl
from jax.experimental.pallas import tpu as pltpu
from jax.experimental.pallas import tpu_sc as plsc

_B = 16384
_F = 26
_V = 100000
_D = 16
_C = 13
_H = 512
_O = 10
_EPS = 1e-5

_NC, _NS = 2, 16
_NW = _NC * _NS
_NR = _F * _D
_RPW = _NR // _NW
_HB = _B // 2


def _sc_gather_t(table, idx_t):
    mesh = plsc.VectorSubcoreMesh(core_axis_name="c", subcore_axis_name="s")

    @functools.partial(
        pl.kernel,
        out_type=jax.ShapeDtypeStruct((_NR, _B), jnp.float32),
        mesh=mesh,
        scratch_types=[
            pltpu.VMEM((_V,), jnp.float32),
            pltpu.VMEM((_B,), jnp.int32),
            pltpu.VMEM((_HB,), jnp.float32),
        ],
        compiler_params=pltpu.CompilerParams(
            use_tc_tiling_on_sc=True, needs_layout_passes=False),
    )
    def k(table_hbm, idx_hbm, out_hbm, row_v, idx_v, out_v):
        wid = lax.axis_index("s") * _NC + lax.axis_index("c")

        def do_row(r, _):
            j = wid * _RPW + r
            f = j // _D
            pltpu.sync_copy(table_hbm.at[j], row_v)
            pltpu.sync_copy(idx_hbm.at[f], idx_v)

            def do_half(h, _):
                @plsc.parallel_loop(0, _HB, step=16, unroll=8)
                def gat(i):
                    iv = idx_v[pl.ds(h * _HB + i, 16)]
                    out_v[pl.ds(i, 16)] = plsc.load_gather(row_v, [iv])

                pltpu.sync_copy(out_v, out_hbm.at[j, pl.ds(h * _HB, _HB)])
                return ()

            lax.fori_loop(0, 2, do_half, (), unroll=True)
            return ()

        lax.fori_loop(0, _RPW, do_row, (), unroll=False)

    return k(table, idx_t)


_R = 512
_G = _B // _R


def _xcstat_body(xc_ref, gc_ref, bc_ref, stat_ref):
    xc = xc_ref[...]
    m = jnp.mean(xc, axis=0, keepdims=True)
    v = jnp.mean(xc * xc, axis=0, keepdims=True) - m * m
    scale = gc_ref[...] * lax.rsqrt(v + _EPS)
    shift = bc_ref[...] - m * scale
    stat_ref[0:1, :] = scale
    stat_ref[1:2, :] = shift


def _l1_body(et_ref, xc_ref, stat_ref, w1e_ref, w1c_ref, b1_ref,
             a1_ref, s1_ref):
    i = pl.program_id(0)
    xcn = xc_ref[...] * stat_ref[0:1, :] + stat_ref[1:2, :]
    h = lax.dot_general(et_ref[...], w1e_ref[...], (((0,), (0,)), ((), ())),
                        preferred_element_type=jnp.float32)
    h += jnp.dot(xcn, w1c_ref[...], preferred_element_type=jnp.float32)
    a1 = jnp.maximum(h + b1_ref[...], 0.0)
    a1_ref[...] = a1

    @pl.when(i == 0)
    def _():
        s1_ref[...] = jnp.zeros_like(s1_ref)

    s1_ref[0:1, :] += jnp.sum(a1, axis=0, keepdims=True)
    s1_ref[1:2, :] += jnp.sum(a1 * a1, axis=0, keepdims=True)


def _l2_body(a1_ref, s1_ref, g1_ref, bt1_ref, w2_ref, b2_ref,
             a2_ref, s2_ref):
    i = pl.program_id(0)
    m = s1_ref[0:1, :] * (1.0 / _B)
    v = s1_ref[1:2, :] * (1.0 / _B) - m * m
    scale = g1_ref[...] * lax.rsqrt(v + _EPS)
    shift = bt1_ref[...] - m * scale
    a1n = a1_ref[...] * scale + shift
    a2 = jnp.maximum(
        jnp.dot(a1n, w2_ref[...], preferred_element_type=jnp.float32)
        + b2_ref[...], 0.0)
    a2_ref[...] = a2

    @pl.when(i == 0)
    def _():
        s2_ref[...] = jnp.zeros_like(s2_ref)

    s2_ref[0:1, :] += jnp.sum(a2, axis=0, keepdims=True)
    s2_ref[1:2, :] += jnp.sum(a2 * a2, axis=0, keepdims=True)


def _l3_body(a2_ref, s2_ref, g2_ref, bt2_ref, w3_ref, b3_ref, out_ref):
    m = s2_ref[0:1, :] * (1.0 / _B)
    v = s2_ref[1:2, :] * (1.0 / _B) - m * m
    scale = g2_ref[...] * lax.rsqrt(v + _EPS)
    shift = bt2_ref[...] - m * scale
    a2n = a2_ref[...] * scale + shift
    out_ref[...] = (
        jnp.dot(a2n, w3_ref[...], preferred_element_type=jnp.float32)
        + b3_ref[...])


def _row(x):
    return x.reshape(1, -1)


def kernel(x_cont, x_cat, emb, gamma_c, beta_c, W1, b1, g1, bt1,
           W2, b2, g2, bt2, W3, b3):
    table = emb.transpose(0, 2, 1).reshape(_NR, _V)
    idx_t = x_cat.T

    e_t = _sc_gather_t(table, idx_t)

    full = lambda s: pl.BlockSpec(s, lambda i: (0, 0))
    blk = lambda r, c: pl.BlockSpec((r, c), lambda i: (i, 0))
    cblk = lambda r, c: pl.BlockSpec((r, c), lambda i: (0, i))

    stat_c = pl.pallas_call(
        _xcstat_body,
        in_specs=[
            pl.BlockSpec((_B, _C), lambda: (0, 0)),
            pl.BlockSpec((1, _C), lambda: (0, 0)),
            pl.BlockSpec((1, _C), lambda: (0, 0)),
        ],
        out_specs=pl.BlockSpec((2, _C), lambda: (0, 0)),
        out_shape=jax.ShapeDtypeStruct((2, _C), jnp.float32),
    )(x_cont, _row(gamma_c), _row(beta_c))

    a1, s1 = pl.pallas_call(
        _l1_body,
        grid=(_G,),
        in_specs=[
            cblk(_NR, _R),
            blk(_R, _C),
            full((2, _C)),
            full((_NR, _H)),
            full((_C, _H)),
            full((1, _H)),
        ],
        out_specs=[blk(_R, _H), full((2, _H))],
        out_shape=[
            jax.ShapeDtypeStruct((_B, _H), jnp.float32),
            jax.ShapeDtypeStruct((2, _H), jnp.float32),
        ],
    )(e_t, x_cont, stat_c, W1[:_NR], W1[_NR:], _row(b1))

    a2, s2 = pl.pallas_call(
        _l2_body,
        grid=(_G,),
        in_specs=[
            blk(_R, _H),
            full((2, _H)),
            full((1, _H)),
            full((1, _H)),
            full((_H, _H // 2)),
            full((1, _H // 2)),
        ],
        out_specs=[blk(_R, _H // 2), full((2, _H // 2))],
        out_shape=[
            jax.ShapeDtypeStruct((_B, _H // 2), jnp.float32),
            jax.ShapeDtypeStruct((2, _H // 2), jnp.float32),
        ],
    )(a1, s1, _row(g1), _row(bt1), W2, _row(b2))

    out = pl.pallas_call(
        _l3_body,
        grid=(_G,),
        in_specs=[
            blk(_R, _H // 2),
            full((2, _H // 2)),
            full((1, _H // 2)),
            full((1, _H // 2)),
            full((_H // 2, _O)),
            full((1, _O)),
        ],
        out_specs=blk(_R, _O),
        out_shape=jax.ShapeDtypeStruct((_B, _O), jnp.float32),
    )(a2, s2, _row(g2), _row(bt2), W3, _row(b3))

    return out


# speedup vs baseline: 47.0962x; 1.2534x over previous
import functools

import jax
import jax.numpy as jnp
from jax import lax
from jax.experimental import pallas as pl
from jax.experimental.pallas import tpu as pltpu
from jax.experimental.pallas import tpu_sc as plsc

_B = 16384
_F = 26
_V = 100000
_D = 16
_C = 13
_H = 512
_O = 10
_EPS = 1e-5

_NC, _NS = 2, 16
_NW = _NC * _NS
_NR = _F * _D
_RPW = _NR // _NW
_HB = _B // 2


def _sc_gather_t(table, idx_t):
    mesh = plsc.VectorSubcoreMesh(core_axis_name="c", subcore_axis_name="s")

    @functools.partial(
        pl.kernel,
        out_type=jax.ShapeDtypeStruct((_NR, _B), jnp.float32),
        mesh=mesh,
        scratch_types=[
            pltpu.VMEM((_V,), jnp.float32),
            pltpu.VMEM((_B,), jnp.int32),
            pltpu.VMEM((_HB,), jnp.float32),
        ],
        compiler_params=pltpu.CompilerParams(
            use_tc_tiling_on_sc=True, needs_layout_passes=False),
    )
    def k(table_hbm, idx_hbm, out_hbm, row_v, idx_v, out_v):
        wid = lax.axis_index("s") * _NC + lax.axis_index("c")

        def do_row(r, f_prev):
            j = wid * _RPW + r
            f = j // _D
            pltpu.sync_copy(table_hbm.at[j], row_v)

            @pl.when(f != f_prev)
            def _():
                pltpu.sync_copy(idx_hbm.at[f], idx_v)

            def do_half(h, _):
                @plsc.parallel_loop(0, _HB, step=16, unroll=8)
                def gat(i):
                    iv = idx_v[pl.ds(h * _HB + i, 16)]
                    out_v[pl.ds(i, 16)] = plsc.load_gather(row_v, [iv])

                pltpu.sync_copy(out_v, out_hbm.at[j, pl.ds(h * _HB, _HB)])
                return ()

            lax.fori_loop(0, 2, do_half, (), unroll=True)
            return f

        lax.fori_loop(0, _RPW, do_row, jnp.int32(-1), unroll=False)

    return k(table, idx_t)


_R = 512
_G = _B // _R


def _xcstat_body(xc_ref, gc_ref, bc_ref, stat_ref):
    xc = xc_ref[...]
    m = jnp.mean(xc, axis=0, keepdims=True)
    v = jnp.mean(xc * xc, axis=0, keepdims=True) - m * m
    scale = gc_ref[...] * lax.rsqrt(v + _EPS)
    shift = bc_ref[...] - m * scale
    stat_ref[0:1, :] = scale
    stat_ref[1:2, :] = shift


def _bn_coefs(s_ref, g_ref, bt_ref):
    m = s_ref[0:1, :] * (1.0 / _B)
    v = s_ref[1:2, :] * (1.0 / _B) - m * m
    scale = g_ref[...] * lax.rsqrt(v + _EPS)
    shift = bt_ref[...] - m * scale
    return scale, shift


def _mlp_body(et_ref, xc_ref, stat_ref, w1e_ref, w1c_ref, b1_ref,
              g1_ref, bt1_ref, w2_ref, b2_ref, g2_ref, bt2_ref,
              w3_ref, b3_ref, out_ref, a1_ref, a2_ref, s1_ref, s2_ref):
    t = pl.program_id(0)
    i = pl.program_id(1)

    @pl.when(t == 0)
    def _():
        xcn = xc_ref[...] * stat_ref[0:1, :] + stat_ref[1:2, :]
        h = lax.dot_general(et_ref[...], w1e_ref[...],
                            (((0,), (0,)), ((), ())),
                            preferred_element_type=jnp.float32)
        h += jnp.dot(xcn, w1c_ref[...], preferred_element_type=jnp.float32)
        a1 = jnp.maximum(h + b1_ref[...], 0.0)
        a1_ref[pl.ds(i * _R, _R), :] = a1

        @pl.when(i == 0)
        def _():
            s1_ref[...] = jnp.zeros_like(s1_ref)

        s1_ref[0:1, :] += jnp.sum(a1, axis=0, keepdims=True)
        s1_ref[1:2, :] += jnp.sum(a1 * a1, axis=0, keepdims=True)

    @pl.when(t == 1)
    def _():
        scale, shift = _bn_coefs(s1_ref, g1_ref, bt1_ref)
        a1n = a1_ref[pl.ds(i * _R, _R), :] * scale + shift
        a2 = jnp.maximum(
            jnp.dot(a1n, w2_ref[...], preferred_element_type=jnp.float32)
            + b2_ref[...], 0.0)
        a2_ref[pl.ds(i * _R, _R), :] = a2

        @pl.when(i == 0)
        def _():
            s2_ref[...] = jnp.zeros_like(s2_ref)

        s2_ref[0:1, :] += jnp.sum(a2, axis=0, keepdims=True)
        s2_ref[1:2, :] += jnp.sum(a2 * a2, axis=0, keepdims=True)

    @pl.when(t == 2)
    def _():
        scale, shift = _bn_coefs(s2_ref, g2_ref, bt2_ref)
        a2n = a2_ref[pl.ds(i * _R, _R), :] * scale + shift
        out_ref[...] = (
            jnp.dot(a2n, w3_ref[...], preferred_element_type=jnp.float32)
            + b3_ref[...])


def _row(x):
    return x.reshape(1, -1)


def kernel(x_cont, x_cat, emb, gamma_c, beta_c, W1, b1, g1, bt1,
           W2, b2, g2, bt2, W3, b3):
    table = emb.transpose(0, 2, 1).reshape(_NR, _V)
    idx_t = x_cat.T

    e_t = _sc_gather_t(table, idx_t)

    stat_c = pl.pallas_call(
        _xcstat_body,
        in_specs=[
            pl.BlockSpec((_B, _C), lambda: (0, 0)),
            pl.BlockSpec((1, _C), lambda: (0, 0)),
            pl.BlockSpec((1, _C), lambda: (0, 0)),
        ],
        out_specs=pl.BlockSpec((2, _C), lambda: (0, 0)),
        out_shape=jax.ShapeDtypeStruct((2, _C), jnp.float32),
    )(x_cont, _row(gamma_c), _row(beta_c))

    full = lambda s: pl.BlockSpec(s, lambda t, i: tuple(0 for _ in s))
    p0blk = lambda r, c: pl.BlockSpec((r, c), lambda t, i: (i * (t == 0), 0))
    et_spec = pl.BlockSpec((_NR, _R), lambda t, i: (0, i * (t == 0)))

    out = pl.pallas_call(
        _mlp_body,
        grid=(3, _G),
        in_specs=[
            et_spec,
            p0blk(_R, _C),
            full((2, _C)),
            full((_NR, _H)),
            full((_C, _H)),
            full((1, _H)),
            full((1, _H)),
            full((1, _H)),
            full((_H, _H // 2)),
            full((1, _H // 2)),
            full((1, _H // 2)),
            full((1, _H // 2)),
            full((_H // 2, _O)),
            full((1, _O)),
        ],
        out_specs=pl.BlockSpec((_R, _O), lambda t, i: (i * (t == 2), 0)),
        out_shape=jax.ShapeDtypeStruct((_B, _O), jnp.float32),
        scratch_shapes=[
            pltpu.VMEM((_B, _H), jnp.float32),
            pltpu.VMEM((_B, _H // 2), jnp.float32),
            pltpu.VMEM((2, _H), jnp.float32),
            pltpu.VMEM((2, _H // 2), jnp.float32),
        ],
        compiler_params=pltpu.CompilerParams(
            vmem_limit_bytes=100 * 1024 * 1024),
    )(e_t, x_cont, stat_c, W1[:_NR], W1[_NR:], _row(b1),
      _row(g1), _row(bt1), W2, _row(b2), _row(g2), _row(bt2),
      W3, _row(b3))

    return out


# speedup vs baseline: 52.9409x; 1.1241x over previous
import functools

import jax
import jax.numpy as jnp
from jax import lax
from jax.experimental import pallas as pl
from jax.experimental.pallas import tpu as pltpu
from jax.experimental.pallas import tpu_sc as plsc

_B = 16384
_F = 26
_V = 100000
_D = 16
_C = 13
_H = 512
_O = 10
_EPS = 1e-5

_NC, _NS = 2, 16
_NW = _NC * _NS
_NR = _F * _D
_RPW = _NR // _NW
_HB = _B // 2


def _sc_gather_t(table, idx_t):
    mesh = plsc.VectorSubcoreMesh(core_axis_name="c", subcore_axis_name="s")

    @functools.partial(
        pl.kernel,
        out_type=jax.ShapeDtypeStruct((_NR, _B), jnp.float32),
        mesh=mesh,
        scratch_types=[
            pltpu.VMEM((_V,), jnp.float32),
            pltpu.VMEM((_B,), jnp.int32),
            pltpu.VMEM((_HB,), jnp.float32),
        ],
        compiler_params=pltpu.CompilerParams(
            use_tc_tiling_on_sc=True, needs_layout_passes=False),
    )
    def k(table_hbm, idx_hbm, out_hbm, row_v, idx_v, out_v):
        wid = lax.axis_index("s") * _NC + lax.axis_index("c")

        def do_row(r, f_prev):
            j = wid * _RPW + r
            f = j // _D
            pltpu.sync_copy(table_hbm.at[j], row_v)

            @pl.when(f != f_prev)
            def _():
                pltpu.sync_copy(idx_hbm.at[f], idx_v)

            def do_half(h, _):
                @plsc.parallel_loop(0, _HB, step=16, unroll=8)
                def gat(i):
                    iv = idx_v[pl.ds(h * _HB + i, 16)]
                    out_v[pl.ds(i, 16)] = plsc.load_gather(row_v, [iv])

                pltpu.sync_copy(out_v, out_hbm.at[j, pl.ds(h * _HB, _HB)])
                return ()

            lax.fori_loop(0, 2, do_half, (), unroll=True)
            return f

        lax.fori_loop(0, _RPW, do_row, jnp.int32(-1), unroll=False)

    return k(table, idx_t)


_R = 1024
_G = _B // _R


def _xcstat_body(xc_ref, gc_ref, bc_ref, stat_ref):
    xc = xc_ref[...]
    m = jnp.mean(xc, axis=0, keepdims=True)
    v = jnp.mean(xc * xc, axis=0, keepdims=True) - m * m
    scale = gc_ref[...] * lax.rsqrt(v + _EPS)
    shift = bc_ref[...] - m * scale
    stat_ref[0:1, :] = scale
    stat_ref[1:2, :] = shift


def _bn_coefs(s_ref, g_ref, bt_ref):
    m = s_ref[0:1, :] * (1.0 / _B)
    v = s_ref[1:2, :] * (1.0 / _B) - m * m
    scale = g_ref[...] * lax.rsqrt(v + _EPS)
    shift = bt_ref[...] - m * scale
    return scale, shift


def _mlp_body(et_ref, xc_ref, stat_ref, w1e_ref, w1c_ref, b1_ref,
              g1_ref, bt1_ref, w2_ref, b2_ref, g2_ref, bt2_ref,
              w3_ref, b3_ref, out_ref, a1_ref, a2_ref, s1_ref, s2_ref):
    t = pl.program_id(0)
    i = pl.program_id(1)

    @pl.when(t == 0)
    def _():
        xcn = xc_ref[...] * stat_ref[0:1, :] + stat_ref[1:2, :]
        h = lax.dot_general(et_ref[...], w1e_ref[...],
                            (((0,), (0,)), ((), ())),
                            preferred_element_type=jnp.float32)
        h += jnp.dot(xcn, w1c_ref[...], preferred_element_type=jnp.float32)
        a1 = jnp.maximum(h + b1_ref[...], 0.0)
        a1_ref[pl.ds(i * _R, _R), :] = a1

        @pl.when(i == 0)
        def _():
            s1_ref[...] = jnp.zeros_like(s1_ref)

        s1_ref[0:1, :] += jnp.sum(a1, axis=0, keepdims=True)
        s1_ref[1:2, :] += jnp.sum(a1 * a1, axis=0, keepdims=True)

    @pl.when(t == 1)
    def _():
        scale, shift = _bn_coefs(s1_ref, g1_ref, bt1_ref)
        a1n = a1_ref[pl.ds(i * _R, _R), :] * scale + shift
        a2 = jnp.maximum(
            jnp.dot(a1n, w2_ref[...], preferred_element_type=jnp.float32)
            + b2_ref[...], 0.0)
        a2_ref[pl.ds(i * _R, _R), :] = a2

        @pl.when(i == 0)
        def _():
            s2_ref[...] = jnp.zeros_like(s2_ref)

        s2_ref[0:1, :] += jnp.sum(a2, axis=0, keepdims=True)
        s2_ref[1:2, :] += jnp.sum(a2 * a2, axis=0, keepdims=True)

    @pl.when(t == 2)
    def _():
        scale, shift = _bn_coefs(s2_ref, g2_ref, bt2_ref)
        a2n = a2_ref[pl.ds(i * _R, _R), :] * scale + shift
        out_ref[...] = (
            jnp.dot(a2n, w3_ref[...], preferred_element_type=jnp.float32)
            + b3_ref[...])


def _row(x):
    return x.reshape(1, -1)


def kernel(x_cont, x_cat, emb, gamma_c, beta_c, W1, b1, g1, bt1,
           W2, b2, g2, bt2, W3, b3):
    table = emb.transpose(0, 2, 1).reshape(_NR, _V)
    idx_t = x_cat.T

    e_t = _sc_gather_t(table, idx_t)

    stat_c = pl.pallas_call(
        _xcstat_body,
        in_specs=[
            pl.BlockSpec((_B, _C), lambda: (0, 0)),
            pl.BlockSpec((1, _C), lambda: (0, 0)),
            pl.BlockSpec((1, _C), lambda: (0, 0)),
        ],
        out_specs=pl.BlockSpec((2, _C), lambda: (0, 0)),
        out_shape=jax.ShapeDtypeStruct((2, _C), jnp.float32),
    )(x_cont, _row(gamma_c), _row(beta_c))

    full = lambda s: pl.BlockSpec(s, lambda t, i: tuple(0 for _ in s))
    p0blk = lambda r, c: pl.BlockSpec((r, c), lambda t, i: (i * (t == 0), 0))
    et_spec = pl.BlockSpec((_NR, _R), lambda t, i: (0, i * (t == 0)))

    out = pl.pallas_call(
        _mlp_body,
        grid=(3, _G),
        in_specs=[
            et_spec,
            p0blk(_R, _C),
            full((2, _C)),
            full((_NR, _H)),
            full((_C, _H)),
            full((1, _H)),
            full((1, _H)),
            full((1, _H)),
            full((_H, _H // 2)),
            full((1, _H // 2)),
            full((1, _H // 2)),
            full((1, _H // 2)),
            full((_H // 2, _O)),
            full((1, _O)),
        ],
        out_specs=pl.BlockSpec((_R, _O), lambda t, i: (i * (t == 2), 0)),
        out_shape=jax.ShapeDtypeStruct((_B, _O), jnp.float32),
        scratch_shapes=[
            pltpu.VMEM((_B, _H), jnp.float32),
            pltpu.VMEM((_B, _H // 2), jnp.float32),
            pltpu.VMEM((2, _H), jnp.float32),
            pltpu.VMEM((2, _H // 2), jnp.float32),
        ],
        compiler_params=pltpu.CompilerParams(
            vmem_limit_bytes=100 * 1024 * 1024),
    )(e_t, x_cont, stat_c, W1[:_NR], W1[_NR:], _row(b1),
      _row(g1), _row(bt1), W2, _row(b2), _row(g2), _row(bt2),
      W3, _row(b3))

    return out


# speedup vs baseline: 55.4750x; 1.0479x over previous
import functools

import jax
import jax.numpy as jnp
from jax import lax
from jax.experimental import pallas as pl
from jax.experimental.pallas import tpu as pltpu
from jax.experimental.pallas import tpu_sc as plsc

_B = 16384
_F = 26
_V = 100000
_D = 16
_C = 13
_H = 512
_O = 10
_EPS = 1e-5

_NC, _NS = 2, 16
_NW = _NC * _NS
_NR = _F * _D
_RPW = _NR // _NW
_QB = _B // 4


def _sc_gather_t(table, idx_t):
    mesh = plsc.VectorSubcoreMesh(core_axis_name="c", subcore_axis_name="s")

    @functools.partial(
        pl.kernel,
        out_type=jax.ShapeDtypeStruct((_NR, _B), jnp.float32),
        mesh=mesh,
        scratch_types=[
            pltpu.VMEM((_V,), jnp.float32),
            pltpu.VMEM((_B,), jnp.int32),
            pltpu.VMEM((_QB,), jnp.float32),
            pltpu.VMEM((_QB,), jnp.float32),
            pltpu.SemaphoreType.DMA,
            pltpu.SemaphoreType.DMA,
        ],
        compiler_params=pltpu.CompilerParams(
            use_tc_tiling_on_sc=True, needs_layout_passes=False),
    )
    def k(table_hbm, idx_hbm, out_hbm, row_v, idx_v, o_a, o_b, sem_a, sem_b):
        wid = lax.axis_index("s") * _NC + lax.axis_index("c")

        def gather_q(j, q, buf):
            @plsc.parallel_loop(0, _QB, step=16, unroll=8)
            def gat(i):
                iv = idx_v[pl.ds(q * _QB + i, 16)]
                buf[pl.ds(i, 16)] = plsc.load_gather(row_v, [iv])

        def do_row(r, f_prev):
            j = wid * _RPW + r
            f = j // _D
            pltpu.sync_copy(table_hbm.at[j], row_v)

            @pl.when(f != f_prev)
            def _():
                pltpu.sync_copy(idx_hbm.at[f], idx_v)

            @pl.when(r > 0)
            def _():
                pltpu.make_async_copy(
                    out_hbm.at[j, pl.ds(2 * _QB, _QB)], o_a, sem_a).wait()
                pltpu.make_async_copy(
                    out_hbm.at[j, pl.ds(3 * _QB, _QB)], o_b, sem_b).wait()

            gather_q(j, 0, o_a)
            h_a0 = pltpu.async_copy(
                o_a, out_hbm.at[j, pl.ds(0 * _QB, _QB)], sem_a)
            gather_q(j, 1, o_b)
            h_b0 = pltpu.async_copy(
                o_b, out_hbm.at[j, pl.ds(1 * _QB, _QB)], sem_b)
            h_a0.wait()
            gather_q(j, 2, o_a)
            pltpu.async_copy(o_a, out_hbm.at[j, pl.ds(2 * _QB, _QB)], sem_a)
            h_b0.wait()
            gather_q(j, 3, o_b)
            pltpu.async_copy(o_b, out_hbm.at[j, pl.ds(3 * _QB, _QB)], sem_b)
            return f

        lax.fori_loop(0, _RPW, do_row, jnp.int32(-1), unroll=False)
        last = wid * _RPW + _RPW - 1
        pltpu.make_async_copy(
            out_hbm.at[last, pl.ds(2 * _QB, _QB)], o_a, sem_a).wait()
        pltpu.make_async_copy(
            out_hbm.at[last, pl.ds(3 * _QB, _QB)], o_b, sem_b).wait()

    return k(table, idx_t)


_R = 1024
_G = _B // _R


def _xcstat_body(xc_ref, gc_ref, bc_ref, stat_ref):
    xc = xc_ref[...]
    m = jnp.mean(xc, axis=0, keepdims=True)
    v = jnp.mean(xc * xc, axis=0, keepdims=True) - m * m
    scale = gc_ref[...] * lax.rsqrt(v + _EPS)
    shift = bc_ref[...] - m * scale
    stat_ref[0:1, :] = scale
    stat_ref[1:2, :] = shift


def _bn_coefs(s_ref, g_ref, bt_ref):
    m = s_ref[0:1, :] * (1.0 / _B)
    v = s_ref[1:2, :] * (1.0 / _B) - m * m
    scale = g_ref[...] * lax.rsqrt(v + _EPS)
    shift = bt_ref[...] - m * scale
    return scale, shift


def _mlp_body(et_ref, xc_ref, stat_ref, w1e_ref, w1c_ref, b1_ref,
              g1_ref, bt1_ref, w2_ref, b2_ref, g2_ref, bt2_ref,
              w3_ref, b3_ref, out_ref, a1_ref, a2_ref, s1_ref, s2_ref):
    t = pl.program_id(0)
    i = pl.program_id(1)

    @pl.when(t == 0)
    def _():
        xcn = xc_ref[...] * stat_ref[0:1, :] + stat_ref[1:2, :]
        h = lax.dot_general(et_ref[...], w1e_ref[...],
                            (((0,), (0,)), ((), ())),
                            preferred_element_type=jnp.float32)
        h += jnp.dot(xcn, w1c_ref[...], preferred_element_type=jnp.float32)
        a1 = jnp.maximum(h + b1_ref[...], 0.0)
        a1_ref[pl.ds(i * _R, _R), :] = a1

        @pl.when(i == 0)
        def _():
            s1_ref[...] = jnp.zeros_like(s1_ref)

        s1_ref[0:1, :] += jnp.sum(a1, axis=0, keepdims=True)
        s1_ref[1:2, :] += jnp.sum(a1 * a1, axis=0, keepdims=True)

    @pl.when(t == 1)
    def _():
        scale, shift = _bn_coefs(s1_ref, g1_ref, bt1_ref)
        a1n = a1_ref[pl.ds(i * _R, _R), :] * scale + shift
        a2 = jnp.maximum(
            jnp.dot(a1n, w2_ref[...], preferred_element_type=jnp.float32)
            + b2_ref[...], 0.0)
        a2_ref[pl.ds(i * _R, _R), :] = a2

        @pl.when(i == 0)
        def _():
            s2_ref[...] = jnp.zeros_like(s2_ref)

        s2_ref[0:1, :] += jnp.sum(a2, axis=0, keepdims=True)
        s2_ref[1:2, :] += jnp.sum(a2 * a2, axis=0, keepdims=True)

    @pl.when(t == 2)
    def _():
        scale, shift = _bn_coefs(s2_ref, g2_ref, bt2_ref)
        a2n = a2_ref[pl.ds(i * _R, _R), :] * scale + shift
        out_ref[...] = (
            jnp.dot(a2n, w3_ref[...], preferred_element_type=jnp.float32)
            + b3_ref[...])


def _row(x):
    return x.reshape(1, -1)


def kernel(x_cont, x_cat, emb, gamma_c, beta_c, W1, b1, g1, bt1,
           W2, b2, g2, bt2, W3, b3):
    table = emb.transpose(0, 2, 1).reshape(_NR, _V)
    idx_t = x_cat.T

    e_t = _sc_gather_t(table, idx_t)

    stat_c = pl.pallas_call(
        _xcstat_body,
        in_specs=[
            pl.BlockSpec((_B, _C), lambda: (0, 0)),
            pl.BlockSpec((1, _C), lambda: (0, 0)),
            pl.BlockSpec((1, _C), lambda: (0, 0)),
        ],
        out_specs=pl.BlockSpec((2, _C), lambda: (0, 0)),
        out_shape=jax.ShapeDtypeStruct((2, _C), jnp.float32),
    )(x_cont, _row(gamma_c), _row(beta_c))

    full = lambda s: pl.BlockSpec(s, lambda t, i: tuple(0 for _ in s))
    p0blk = lambda r, c: pl.BlockSpec((r, c), lambda t, i: (i * (t == 0), 0))
    et_spec = pl.BlockSpec((_NR, _R), lambda t, i: (0, i * (t == 0)))

    out = pl.pallas_call(
        _mlp_body,
        grid=(3, _G),
        in_specs=[
            et_spec,
            p0blk(_R, _C),
            full((2, _C)),
            full((_NR, _H)),
            full((_C, _H)),
            full((1, _H)),
            full((1, _H)),
            full((1, _H)),
            full((_H, _H // 2)),
            full((1, _H // 2)),
            full((1, _H // 2)),
            full((1, _H // 2)),
            full((_H // 2, _O)),
            full((1, _O)),
        ],
        out_specs=pl.BlockSpec((_R, _O), lambda t, i: (i * (t == 2), 0)),
        out_shape=jax.ShapeDtypeStruct((_B, _O), jnp.float32),
        scratch_shapes=[
            pltpu.VMEM((_B, _H), jnp.float32),
            pltpu.VMEM((_B, _H // 2), jnp.float32),
            pltpu.VMEM((2, _H), jnp.float32),
            pltpu.VMEM((2, _H // 2), jnp.float32),
        ],
        compiler_params=pltpu.CompilerParams(
            vmem_limit_bytes=100 * 1024 * 1024),
    )(e_t, x_cont, stat_c, W1[:_NR], W1[_NR:], _row(b1),
      _row(g1), _row(bt1), W2, _row(b2), _row(g2), _row(bt2),
      W3, _row(b3))

    return out


# speedup vs baseline: 55.5332x; 1.0011x over previous
import functools

import jax
import jax.numpy as jnp
from jax import lax
from jax.experimental import pallas as pl
from jax.experimental.pallas import tpu as pltpu
from jax.experimental.pallas import tpu_sc as plsc

_B = 16384
_F = 26
_V = 100000
_D = 16
_C = 13
_H = 512
_O = 10
_EPS = 1e-5

_NC, _NS = 2, 16
_NW = _NC * _NS
_NR = _F * _D
_RPW = _NR // _NW
_QB = _B // 4


def _sc_gather_t(table, idx_t):
    mesh = plsc.VectorSubcoreMesh(core_axis_name="c", subcore_axis_name="s")

    @functools.partial(
        pl.kernel,
        out_type=jax.ShapeDtypeStruct((_NR, _B), jnp.float32),
        mesh=mesh,
        scratch_types=[
            pltpu.VMEM((_V,), jnp.float32),
            pltpu.VMEM((_B,), jnp.int32),
            pltpu.VMEM((_QB,), jnp.float32),
            pltpu.VMEM((_QB,), jnp.float32),
            pltpu.SemaphoreType.DMA,
            pltpu.SemaphoreType.DMA,
        ],
        compiler_params=pltpu.CompilerParams(
            use_tc_tiling_on_sc=True, needs_layout_passes=False),
    )
    def k(table_hbm, idx_hbm, out_hbm, row_v, idx_v, o_a, o_b, sem_a, sem_b):
        wid = lax.axis_index("s") * _NC + lax.axis_index("c")

        def gather_q(j, q, buf):
            @plsc.parallel_loop(0, _QB, step=16, unroll=16)
            def gat(i):
                iv = idx_v[pl.ds(q * _QB + i, 16)]
                buf[pl.ds(i, 16)] = plsc.load_gather(row_v, [iv])

        def do_row(r, f_prev):
            j = wid * _RPW + r
            f = j // _D
            pltpu.sync_copy(table_hbm.at[j], row_v)

            @pl.when(f != f_prev)
            def _():
                pltpu.sync_copy(idx_hbm.at[f], idx_v)

            @pl.when(r > 0)
            def _():
                pltpu.make_async_copy(
                    out_hbm.at[j, pl.ds(2 * _QB, _QB)], o_a, sem_a).wait()
                pltpu.make_async_copy(
                    out_hbm.at[j, pl.ds(3 * _QB, _QB)], o_b, sem_b).wait()

            gather_q(j, 0, o_a)
            h_a0 = pltpu.async_copy(
                o_a, out_hbm.at[j, pl.ds(0 * _QB, _QB)], sem_a)
            gather_q(j, 1, o_b)
            h_b0 = pltpu.async_copy(
                o_b, out_hbm.at[j, pl.ds(1 * _QB, _QB)], sem_b)
            h_a0.wait()
            gather_q(j, 2, o_a)
            pltpu.async_copy(o_a, out_hbm.at[j, pl.ds(2 * _QB, _QB)], sem_a)
            h_b0.wait()
            gather_q(j, 3, o_b)
            pltpu.async_copy(o_b, out_hbm.at[j, pl.ds(3 * _QB, _QB)], sem_b)
            return f

        lax.fori_loop(0, _RPW, do_row, jnp.int32(-1), unroll=False)
        last = wid * _RPW + _RPW - 1
        pltpu.make_async_copy(
            out_hbm.at[last, pl.ds(2 * _QB, _QB)], o_a, sem_a).wait()
        pltpu.make_async_copy(
            out_hbm.at[last, pl.ds(3 * _QB, _QB)], o_b, sem_b).wait()

    return k(table, idx_t)


_R = 1024
_G = _B // _R


def _xcstat_body(xc_ref, gc_ref, bc_ref, stat_ref):
    xc = xc_ref[...]
    m = jnp.mean(xc, axis=0, keepdims=True)
    v = jnp.mean(xc * xc, axis=0, keepdims=True) - m * m
    scale = gc_ref[...] * lax.rsqrt(v + _EPS)
    shift = bc_ref[...] - m * scale
    stat_ref[0:1, :] = scale
    stat_ref[1:2, :] = shift


def _bn_coefs(s_ref, g_ref, bt_ref):
    m = s_ref[0:1, :] * (1.0 / _B)
    v = s_ref[1:2, :] * (1.0 / _B) - m * m
    scale = g_ref[...] * lax.rsqrt(v + _EPS)
    shift = bt_ref[...] - m * scale
    return scale, shift


def _mlp_body(et_ref, xc_ref, stat_ref, w1e_ref, w1c_ref, b1_ref,
              g1_ref, bt1_ref, w2_ref, b2_ref, g2_ref, bt2_ref,
              w3_ref, b3_ref, out_ref, a1_ref, a2_ref, s1_ref, s2_ref):
    t = pl.program_id(0)
    i = pl.program_id(1)

    @pl.when(t == 0)
    def _():
        xcn = xc_ref[...] * stat_ref[0:1, :] + stat_ref[1:2, :]
        h = lax.dot_general(et_ref[...], w1e_ref[...],
                            (((0,), (0,)), ((), ())),
                            preferred_element_type=jnp.float32)
        h += jnp.dot(xcn, w1c_ref[...], preferred_element_type=jnp.float32)
        a1 = jnp.maximum(h + b1_ref[...], 0.0)
        a1_ref[pl.ds(i * _R, _R), :] = a1

        @pl.when(i == 0)
        def _():
            s1_ref[...] = jnp.zeros_like(s1_ref)

        s1_ref[0:1, :] += jnp.sum(a1, axis=0, keepdims=True)
        s1_ref[1:2, :] += jnp.sum(a1 * a1, axis=0, keepdims=True)

    @pl.when(t == 1)
    def _():
        scale, shift = _bn_coefs(s1_ref, g1_ref, bt1_ref)
        a1n = a1_ref[pl.ds(i * _R, _R), :] * scale + shift
        a2 = jnp.maximum(
            jnp.dot(a1n, w2_ref[...], preferred_element_type=jnp.float32)
            + b2_ref[...], 0.0)
        a2_ref[pl.ds(i * _R, _R), :] = a2

        @pl.when(i == 0)
        def _():
            s2_ref[...] = jnp.zeros_like(s2_ref)

        s2_ref[0:1, :] += jnp.sum(a2, axis=0, keepdims=True)
        s2_ref[1:2, :] += jnp.sum(a2 * a2, axis=0, keepdims=True)

    @pl.when(t == 2)
    def _():
        scale, shift = _bn_coefs(s2_ref, g2_ref, bt2_ref)
        a2n = a2_ref[pl.ds(i * _R, _R), :] * scale + shift
        out_ref[...] = (
            jnp.dot(a2n, w3_ref[...], preferred_element_type=jnp.float32)
            + b3_ref[...])


def _row(x):
    return x.reshape(1, -1)


def kernel(x_cont, x_cat, emb, gamma_c, beta_c, W1, b1, g1, bt1,
           W2, b2, g2, bt2, W3, b3):
    table = emb.transpose(0, 2, 1).reshape(_NR, _V)
    idx_t = x_cat.T

    e_t = _sc_gather_t(table, idx_t)

    stat_c = pl.pallas_call(
        _xcstat_body,
        in_specs=[
            pl.BlockSpec((_B, _C), lambda: (0, 0)),
            pl.BlockSpec((1, _C), lambda: (0, 0)),
            pl.BlockSpec((1, _C), lambda: (0, 0)),
        ],
        out_specs=pl.BlockSpec((2, _C), lambda: (0, 0)),
        out_shape=jax.ShapeDtypeStruct((2, _C), jnp.float32),
    )(x_cont, _row(gamma_c), _row(beta_c))

    full = lambda s: pl.BlockSpec(s, lambda t, i: tuple(0 for _ in s))
    p0blk = lambda r, c: pl.BlockSpec((r, c), lambda t, i: (i * (t == 0), 0))
    et_spec = pl.BlockSpec((_NR, _R), lambda t, i: (0, i * (t == 0)))

    out = pl.pallas_call(
        _mlp_body,
        grid=(3, _G),
        in_specs=[
            et_spec,
            p0blk(_R, _C),
            full((2, _C)),
            full((_NR, _H)),
            full((_C, _H)),
            full((1, _H)),
            full((1, _H)),
            full((1, _H)),
            full((_H, _H // 2)),
            full((1, _H // 2)),
            full((1, _H // 2)),
            full((1, _H // 2)),
            full((_H // 2, _O)),
            full((1, _O)),
        ],
        out_specs=pl.BlockSpec((_R, _O), lambda t, i: (i * (t == 2), 0)),
        out_shape=jax.ShapeDtypeStruct((_B, _O), jnp.float32),
        scratch_shapes=[
            pltpu.VMEM((_B, _H), jnp.float32),
            pltpu.VMEM((_B, _H // 2), jnp.float32),
            pltpu.VMEM((2, _H), jnp.float32),
            pltpu.VMEM((2, _H // 2), jnp.float32),
        ],
        compiler_params=pltpu.CompilerParams(
            vmem_limit_bytes=100 * 1024 * 1024),
    )(e_t, x_cont, stat_c, W1[:_NR], W1[_NR:], _row(b1),
      _row(g1), _row(bt1), W2, _row(b2), _row(g2), _row(bt2),
      W3, _row(b3))

    return out
